# Initial kernel scaffold; baseline (speedup 1.0000x reference)
import jax
import jax.numpy as jnp
from jax.experimental import pallas as pl


def kernel(x, edge_index_pa, edge_index_ap, y):
    raise NotImplementedError("write your pallas kernel here")


# speedup vs baseline: 8.3645x; 8.3645x over previous
import jax
import jax.numpy as jnp
from jax import lax
from jax.experimental import pallas as pl
from jax.experimental.pallas import tpu as pltpu
from jax.experimental.pallas import tpu_sc as plsc

N = 10000
E = 320000
D = 128
C = 16

NTILES = 16
LANES = 16

ROWS_PER_TILE = 640
NACC = NTILES * ROWS_PER_TILE
TRASH = N

CHUNK = 128
NCH = 157
EPT = NCH * CHUNK
E_PAD = NTILES * EPT

W = 80


def _vgather16(v, idx):
    return lax.gather(
        v, idx[:, None],
        lax.GatherDimensionNumbers(
            offset_dims=(), collapsed_slice_dims=(0,), start_index_map=(0,)),
        (1,), mode=lax.GatherScatterMode.PROMISE_IN_BOUNDS)


def _hop_body(t0, t1, src3d, dst3d, out0, out1,
              src_idx, dst_idx, rows, ones_b, cz, cntb,
              acc, cnt_sh):
    cid = lax.axis_index("c")
    sid = lax.axis_index("s")
    base_row = sid * ROWS_PER_TILE

    zero16 = jnp.zeros((LANES,), jnp.float32)
    one16 = jnp.ones((LANES,), jnp.float32)
    for r in range(CHUNK):
        for cc in range(W // LANES):
            rows[r, pl.ds(cc * LANES, LANES)] = zero16
    for i in range(CHUNK // LANES):
        ones_b[pl.ds(i * LANES, LANES)] = one16
    for i in range(ROWS_PER_TILE // LANES):
        cz[pl.ds(i * LANES, LANES)] = zero16

    pltpu.sync_copy(cz, cnt_sh.at[pl.ds(base_row, ROWS_PER_TILE)])
    for k in range(ROWS_PER_TILE // CHUNK):
        pltpu.sync_copy(rows, acc.at[pl.ds(base_row + k * CHUNK, CHUNK)])

    pltpu.sync_copy(src3d.at[sid], src_idx)
    pltpu.sync_copy(dst3d.at[sid], dst_idx)

    plsc.subcore_barrier()

    def acc_step(j, carry):
        sij = src_idx.at[j]
        dij = dst_idx.at[j]

        @pl.when(cid == 0)
        def _():
            pltpu.sync_copy(t0.at[sij], rows)

        @pl.when(cid == 1)
        def _():
            pltpu.sync_copy(t1.at[sij], rows)

        pltpu.sync_copy(rows, acc.at[dij], add=True)
        pltpu.sync_copy(ones_b, cnt_sh.at[dij], add=True)
        return carry

    lax.fori_loop(0, NCH, acc_step, 0)

    plsc.subcore_barrier()

    def norm_step(k, carry):
        chunk_base = base_row + k * CHUNK
        pltpu.sync_copy(cnt_sh.at[pl.ds(chunk_base, CHUNK)], cntb)
        pltpu.sync_copy(acc.at[pl.ds(chunk_base, CHUNK)], rows)
        for g in range(CHUNK // LANES):
            c16 = cntb[pl.ds(g * LANES, LANES)]
            rec16 = 1.0 / jnp.maximum(c16, 1.0)
            for p in range(LANES):
                r = g * LANES + p
                rec = _vgather16(rec16, jnp.full((LANES,), p, jnp.int32))
                for cc in range(W // LANES):
                    sl = pl.ds(cc * LANES, LANES)
                    rows[r, sl] = rows[r, sl] * rec

        @pl.when(cid == 0)
        def _():
            pltpu.sync_copy(rows, out0.at[pl.ds(chunk_base, CHUNK)])

        @pl.when(cid == 1)
        def _():
            pltpu.sync_copy(rows, out1.at[pl.ds(chunk_base, CHUNK)])

        return carry

    lax.fori_loop(0, ROWS_PER_TILE // CHUNK, norm_step, 0)


_hop = pl.kernel(
    _hop_body,
    out_type=(
        jax.ShapeDtypeStruct((NACC, W), jnp.float32),
        jax.ShapeDtypeStruct((NACC, W), jnp.float32),
    ),
    mesh=plsc.VectorSubcoreMesh(core_axis_name="c", subcore_axis_name="s"),
    scratch_types=(
        pltpu.VMEM((NCH, CHUNK), jnp.int32),
        pltpu.VMEM((NCH, CHUNK), jnp.int32),
        pltpu.VMEM((CHUNK, W), jnp.float32),
        pltpu.VMEM((CHUNK,), jnp.float32),
        pltpu.VMEM((ROWS_PER_TILE,), jnp.float32),
        pltpu.VMEM((CHUNK,), jnp.float32),
        pltpu.VMEM_SHARED((NACC, W), jnp.float32),
        pltpu.VMEM_SHARED((NACC,), jnp.float32),
    ),
    compiler_params=pltpu.CompilerParams(use_tc_tiling_on_sc=False),
)


def _pad_edges(src, dst):
    npad = E_PAD - E
    pad_src = (jnp.arange(npad, dtype=jnp.int32) % N)
    pad_dst = jnp.full((npad,), TRASH, dtype=jnp.int32)
    s = jnp.concatenate([src, pad_src]).reshape(NTILES, NCH, CHUNK)
    d = jnp.concatenate([dst, pad_dst]).reshape(NTILES, NCH, CHUNK)
    return s, d


def kernel(x, edge_index_pa, edge_index_ap, y):
    rowpad = ((0, NACC - N), (0, 0))
    t0 = jnp.pad(x[:, : W - C], ((0, NACC - N), (0, C)))
    t1 = jnp.pad(jnp.concatenate([x[:, W - C:], y], axis=1), rowpad)

    pa_s, pa_d = _pad_edges(edge_index_pa[0], edge_index_pa[1])
    ap_s, ap_d = _pad_edges(edge_index_ap[0], edge_index_ap[1])

    h0, h1 = _hop(t0, t1, pa_s, pa_d)
    m0, m1 = _hop(h0, h1, ap_s, ap_d)

    nf = W - C
    mp = jnp.concatenate([m0[:N, :nf], m1[:N, :nf]], axis=1)
    lp = m1[:N, nf:]
    return (mp, lp)


# speedup vs baseline: 9.1448x; 1.0933x over previous
import jax
import jax.numpy as jnp
from jax import lax
from jax.experimental import pallas as pl
from jax.experimental.pallas import tpu as pltpu
from jax.experimental.pallas import tpu_sc as plsc

N = 10000
E = 320000
D = 128
C = 16

NTILES = 16
LANES = 16

ROWS_PER_TILE = 640
NACC = NTILES * ROWS_PER_TILE
TRASH = N

CHUNK = 128
NCH = 158
NCHG = NCH + 2
EPT = NCH * CHUNK
E_PAD = NTILES * EPT

W = 80


def _vgather16(v, idx):
    return lax.gather(
        v, idx[:, None],
        lax.GatherDimensionNumbers(
            offset_dims=(), collapsed_slice_dims=(0,), start_index_map=(0,)),
        (1,), mode=lax.GatherScatterMode.PROMISE_IN_BOUNDS)


def _hop_body(t0, t1, src3d, dst3d, out0, out1,
              src_idx, dst_idx, rows0, rows1, ones_b, cz, cntb,
              acc, cnt_sh, sem_g0, sem_g1, sem_s0, sem_s1):
    cid = lax.axis_index("c")
    sid = lax.axis_index("s")
    base_row = sid * ROWS_PER_TILE

    zero16 = jnp.zeros((LANES,), jnp.float32)
    one16 = jnp.ones((LANES,), jnp.float32)
    for r in range(CHUNK):
        for cc in range(W // LANES):
            rows0[r, pl.ds(cc * LANES, LANES)] = zero16
    for i in range(CHUNK // LANES):
        ones_b[pl.ds(i * LANES, LANES)] = one16
    for i in range(ROWS_PER_TILE // LANES):
        cz[pl.ds(i * LANES, LANES)] = zero16

    pltpu.sync_copy(cz, cnt_sh.at[pl.ds(base_row, ROWS_PER_TILE)])
    for k in range(ROWS_PER_TILE // CHUNK):
        pltpu.sync_copy(rows0, acc.at[pl.ds(base_row + k * CHUNK, CHUNK)])

    pltpu.sync_copy(src3d.at[sid], src_idx)
    pltpu.sync_copy(dst3d.at[sid], dst_idx)

    def gather(j, rows, sem):
        @pl.when(cid == 0)
        def _():
            pltpu.async_copy(t0.at[src_idx.at[j]], rows, sem)

        @pl.when(cid == 1)
        def _():
            pltpu.async_copy(t1.at[src_idx.at[j]], rows, sem)

    def gather_wait(rows, sem):
        pltpu.make_async_copy(t0.at[src_idx.at[0]], rows, sem).wait()

    gather(0, rows0, sem_g0)
    gather(1, rows1, sem_g1)

    plsc.subcore_barrier()

    def acc_step(i, carry):
        for b, (rows, sem_g, sem_s) in enumerate(
                ((rows0, sem_g0, sem_s0), (rows1, sem_g1, sem_s1))):
            j = 2 * i + b
            dij = dst_idx.at[j]
            gather_wait(rows, sem_g)
            rd = pltpu.async_copy(rows, acc.at[dij], sem_s, add=True)
            od = pltpu.async_copy(ones_b, cnt_sh.at[dij], sem_s, add=True)
            rd.wait()
            od.wait()
            gather(j + 2, rows, sem_g)
        return carry

    lax.fori_loop(0, NCH // 2, acc_step, 0)

    gather_wait(rows0, sem_g0)
    gather_wait(rows1, sem_g1)

    plsc.subcore_barrier()

    def norm_step(k, carry):
        chunk_base = base_row + k * CHUNK
        pltpu.sync_copy(cnt_sh.at[pl.ds(chunk_base, CHUNK)], cntb)
        pltpu.sync_copy(acc.at[pl.ds(chunk_base, CHUNK)], rows0)
        for g in range(CHUNK // LANES):
            c16 = cntb[pl.ds(g * LANES, LANES)]
            rec16 = 1.0 / jnp.maximum(c16, 1.0)
            for p in range(LANES):
                r = g * LANES + p
                rec = _vgather16(rec16, jnp.full((LANES,), p, jnp.int32))
                for cc in range(W // LANES):
                    sl = pl.ds(cc * LANES, LANES)
                    rows0[r, sl] = rows0[r, sl] * rec

        @pl.when(cid == 0)
        def _():
            pltpu.sync_copy(rows0, out0.at[pl.ds(chunk_base, CHUNK)])

        @pl.when(cid == 1)
        def _():
            pltpu.sync_copy(rows0, out1.at[pl.ds(chunk_base, CHUNK)])

        return carry

    lax.fori_loop(0, ROWS_PER_TILE // CHUNK, norm_step, 0)


_hop = pl.kernel(
    _hop_body,
    out_type=(
        jax.ShapeDtypeStruct((NACC, W), jnp.float32),
        jax.ShapeDtypeStruct((NACC, W), jnp.float32),
    ),
    mesh=plsc.VectorSubcoreMesh(core_axis_name="c", subcore_axis_name="s"),
    scratch_types=(
        pltpu.VMEM((NCHG, CHUNK), jnp.int32),
        pltpu.VMEM((NCHG, CHUNK), jnp.int32),
        pltpu.VMEM((CHUNK, W), jnp.float32),
        pltpu.VMEM((CHUNK, W), jnp.float32),
        pltpu.VMEM((CHUNK,), jnp.float32),
        pltpu.VMEM((ROWS_PER_TILE,), jnp.float32),
        pltpu.VMEM((CHUNK,), jnp.float32),
        pltpu.VMEM_SHARED((NACC, W), jnp.float32),
        pltpu.VMEM_SHARED((NACC,), jnp.float32),
        pltpu.SemaphoreType.DMA,
        pltpu.SemaphoreType.DMA,
        pltpu.SemaphoreType.DMA,
        pltpu.SemaphoreType.DMA,
    ),
    compiler_params=pltpu.CompilerParams(use_tc_tiling_on_sc=False),
)


def _pad_edges(src, dst):
    npad = E_PAD - E
    pad_src = (jnp.arange(npad, dtype=jnp.int32) % N)
    pad_dst = TRASH + (jnp.arange(npad, dtype=jnp.int32) % (NACC - N))
    s = jnp.concatenate([src, pad_src]).reshape(NTILES, NCH, CHUNK)
    d = jnp.concatenate([dst, pad_dst]).reshape(NTILES, NCH, CHUNK)
    dummy = jnp.zeros((NTILES, 2, CHUNK), jnp.int32)
    return (jnp.concatenate([s, dummy], axis=1),
            jnp.concatenate([d, dummy], axis=1))


def kernel(x, edge_index_pa, edge_index_ap, y):
    rowpad = ((0, NACC - N), (0, 0))
    t0 = jnp.pad(x[:, : W - C], ((0, NACC - N), (0, C)))
    t1 = jnp.pad(jnp.concatenate([x[:, W - C:], y], axis=1), rowpad)

    pa_s, pa_d = _pad_edges(edge_index_pa[0], edge_index_pa[1])
    ap_s, ap_d = _pad_edges(edge_index_ap[0], edge_index_ap[1])

    h0, h1 = _hop(t0, t1, pa_s, pa_d)
    m0, m1 = _hop(h0, h1, ap_s, ap_d)

    nf = W - C
    mp = jnp.concatenate([m0[:N, :nf], m1[:N, :nf]], axis=1)
    lp = m1[:N, nf:]
    return (mp, lp)


# speedup vs baseline: 9.9486x; 1.0879x over previous
import jax
import jax.numpy as jnp
from jax import lax
from jax.experimental import pallas as pl
from jax.experimental.pallas import tpu as pltpu
from jax.experimental.pallas import tpu_sc as plsc

N = 10000
E = 320000
D = 128
C = 16

NTILES = 16
LANES = 16

ROWS_PER_TILE = 640
NACC = NTILES * ROWS_PER_TILE
TRASH = N

CHUNK = 128
NCH = 158
NCHG = NCH + 2
EPT = NCH * CHUNK
E_PAD = NTILES * EPT

W = 80


def _vgather16(v, idx):
    return lax.gather(
        v, idx[:, None],
        lax.GatherDimensionNumbers(
            offset_dims=(), collapsed_slice_dims=(0,), start_index_map=(0,)),
        (1,), mode=lax.GatherScatterMode.PROMISE_IN_BOUNDS)


def _hop_body(t0, t1, src3d, dst3d, out0, out1,
              src_idx, dst_idx, rows0, rows1, rows2, ones_b, cz, cntb,
              acc, cnt_sh,
              sem_g0, sem_g1, sem_g2, sem_s0, sem_s1, sem_s2):
    cid = lax.axis_index("c")
    sid = lax.axis_index("s")
    base_row = sid * ROWS_PER_TILE

    zero16 = jnp.zeros((LANES,), jnp.float32)
    one16 = jnp.ones((LANES,), jnp.float32)
    for r in range(CHUNK):
        for cc in range(W // LANES):
            rows0[r, pl.ds(cc * LANES, LANES)] = zero16
    for i in range(CHUNK // LANES):
        ones_b[pl.ds(i * LANES, LANES)] = one16
    for i in range(ROWS_PER_TILE // LANES):
        cz[pl.ds(i * LANES, LANES)] = zero16

    pltpu.sync_copy(cz, cnt_sh.at[pl.ds(base_row, ROWS_PER_TILE)])
    for k in range(ROWS_PER_TILE // CHUNK):
        pltpu.sync_copy(rows0, acc.at[pl.ds(base_row + k * CHUNK, CHUNK)])

    pltpu.sync_copy(src3d.at[sid], src_idx)
    pltpu.sync_copy(dst3d.at[sid], dst_idx)

    bufs = (rows0, rows1, rows2)
    gsems = (sem_g0, sem_g1, sem_g2)
    ssems = (sem_s0, sem_s1, sem_s2)

    def gather(j, b):
        @pl.when(cid == 0)
        def _():
            pltpu.async_copy(t0.at[src_idx.at[j]], bufs[b], gsems[b])

        @pl.when(cid == 1)
        def _():
            pltpu.async_copy(t1.at[src_idx.at[j]], bufs[b], gsems[b])

    def gather_wait(b):
        pltpu.make_async_copy(t0.at[src_idx.at[0]], bufs[b], gsems[b]).wait()

    def scatter(j, b):
        dij = dst_idx.at[j]
        pltpu.async_copy(bufs[b], acc.at[dij], ssems[b], add=True)
        pltpu.async_copy(ones_b, cnt_sh.at[dij], ssems[b], add=True)

    def scatter_wait(b):
        pltpu.make_async_copy(bufs[b], acc.at[dst_idx.at[0]], ssems[b]).wait()
        pltpu.make_async_copy(ones_b, cnt_sh.at[dst_idx.at[0]], ssems[b]).wait()

    gather(0, 0)
    gather(1, 1)

    plsc.subcore_barrier()

    gather_wait(0)
    scatter(0, 0)
    gather(2, 2)
    gather_wait(1)
    scatter(1, 1)
    scatter_wait(0)
    gather(3, 0)

    def acc_step(i, carry):
        for b in range(3):
            j = 3 * i + 2 + b
            bj = (2 + b) % 3
            gather_wait(bj)
            scatter(j, bj)
            scatter_wait((bj + 2) % 3)
            gather(j + 2, (bj + 2) % 3)
        return carry

    lax.fori_loop(0, (NCH - 2) // 3, acc_step, 0)

    scatter_wait((NCH - 1) % 3)
    gather_wait(NCH % 3)
    gather_wait((NCH + 1) % 3)

    plsc.subcore_barrier()

    def norm_step(k, carry):
        chunk_base = base_row + k * CHUNK
        pltpu.sync_copy(cnt_sh.at[pl.ds(chunk_base, CHUNK)], cntb)
        pltpu.sync_copy(acc.at[pl.ds(chunk_base, CHUNK)], rows0)
        for g in range(CHUNK // LANES):
            c16 = cntb[pl.ds(g * LANES, LANES)]
            rec16 = 1.0 / jnp.maximum(c16, 1.0)
            for p in range(LANES):
                r = g * LANES + p
                rec = _vgather16(rec16, jnp.full((LANES,), p, jnp.int32))
                for cc in range(W // LANES):
                    sl = pl.ds(cc * LANES, LANES)
                    rows0[r, sl] = rows0[r, sl] * rec

        @pl.when(cid == 0)
        def _():
            pltpu.sync_copy(rows0, out0.at[pl.ds(chunk_base, CHUNK)])

        @pl.when(cid == 1)
        def _():
            pltpu.sync_copy(rows0, out1.at[pl.ds(chunk_base, CHUNK)])

        return carry

    lax.fori_loop(0, ROWS_PER_TILE // CHUNK, norm_step, 0)


_hop = pl.kernel(
    _hop_body,
    out_type=(
        jax.ShapeDtypeStruct((NACC, W), jnp.float32),
        jax.ShapeDtypeStruct((NACC, W), jnp.float32),
    ),
    mesh=plsc.VectorSubcoreMesh(core_axis_name="c", subcore_axis_name="s"),
    scratch_types=(
        pltpu.VMEM((NCHG, CHUNK), jnp.int32),
        pltpu.VMEM((NCHG, CHUNK), jnp.int32),
        pltpu.VMEM((CHUNK, W), jnp.float32),
        pltpu.VMEM((CHUNK, W), jnp.float32),
        pltpu.VMEM((CHUNK, W), jnp.float32),
        pltpu.VMEM((CHUNK,), jnp.float32),
        pltpu.VMEM((ROWS_PER_TILE,), jnp.float32),
        pltpu.VMEM((CHUNK,), jnp.float32),
        pltpu.VMEM_SHARED((NACC, W), jnp.float32),
        pltpu.VMEM_SHARED((NACC,), jnp.float32),
        pltpu.SemaphoreType.DMA,
        pltpu.SemaphoreType.DMA,
        pltpu.SemaphoreType.DMA,
        pltpu.SemaphoreType.DMA,
        pltpu.SemaphoreType.DMA,
        pltpu.SemaphoreType.DMA,
    ),
    compiler_params=pltpu.CompilerParams(use_tc_tiling_on_sc=False),
)


def _pad_edges(src, dst):
    npad = E_PAD - E
    pad_src = (jnp.arange(npad, dtype=jnp.int32) % N)
    pad_dst = TRASH + (jnp.arange(npad, dtype=jnp.int32) % (NACC - N))
    s = jnp.concatenate([src, pad_src]).reshape(NTILES, NCH, CHUNK)
    d = jnp.concatenate([dst, pad_dst]).reshape(NTILES, NCH, CHUNK)
    dummy = jnp.zeros((NTILES, 2, CHUNK), jnp.int32)
    return (jnp.concatenate([s, dummy], axis=1),
            jnp.concatenate([d, dummy], axis=1))


def kernel(x, edge_index_pa, edge_index_ap, y):
    rowpad = ((0, NACC - N), (0, 0))
    t0 = jnp.pad(x[:, : W - C], ((0, NACC - N), (0, C)))
    t1 = jnp.pad(jnp.concatenate([x[:, W - C:], y], axis=1), rowpad)

    pa_s, pa_d = _pad_edges(edge_index_pa[0], edge_index_pa[1])
    ap_s, ap_d = _pad_edges(edge_index_ap[0], edge_index_ap[1])

    h0, h1 = _hop(t0, t1, pa_s, pa_d)
    m0, m1 = _hop(h0, h1, ap_s, ap_d)

    nf = W - C
    mp = jnp.concatenate([m0[:N, :nf], m1[:N, :nf]], axis=1)
    lp = m1[:N, nf:]
    return (mp, lp)
